# baseline (speedup 1.0000x reference)
import jax
import jax.numpy as jnp
from jax.experimental import pallas as pl
from jax.experimental.pallas import tpu as pltpu

_P, _C = 96, 128
_BLOCK_ROWS = 128


def _gather_body(idx_ref, table_ref, pe_ref):
    for j in range(_P):
        pe_ref[j, :] = table_ref[idx_ref[j], :]


def _add_body(x_ref, pe_ref, o_ref):
    o_ref[...] = x_ref[...] + pe_ref[...]


def kernel(input_data, index, position_embedding):
    b, n, p, c = input_data.shape
    bn = b * n
    x = input_data.reshape(bn, p, c)

    gather_spec = pltpu.PrefetchScalarGridSpec(
        num_scalar_prefetch=1,
        grid=(1,),
        in_specs=[pl.BlockSpec((1000, c), lambda i, idx_ref: (0, 0))],
        out_specs=pl.BlockSpec((_P, c), lambda i, idx_ref: (0, 0)),
        scratch_shapes=[],
    )
    pe = pl.pallas_call(
        _gather_body,
        grid_spec=gather_spec,
        out_shape=jax.ShapeDtypeStruct((_P, c), jnp.float32),
    )(index.astype(jnp.int32), position_embedding)

    out = pl.pallas_call(
        _add_body,
        grid=(bn // _BLOCK_ROWS,),
        in_specs=[
            pl.BlockSpec((_BLOCK_ROWS, p, c), lambda i: (i, 0, 0)),
            pl.BlockSpec((p, c), lambda i: (0, 0)),
        ],
        out_specs=pl.BlockSpec((_BLOCK_ROWS, p, c), lambda i: (i, 0, 0)),
        out_shape=jax.ShapeDtypeStruct((bn, p, c), jnp.float32),
    )(x, pe)
    return out.reshape(b, n, p, c)
